# Initial kernel scaffold; baseline (speedup 1.0000x reference)
import jax
import jax.numpy as jnp
from jax.experimental import pallas as pl


def kernel(x, edge_index, W1, b1, Wmu, bmu, Wlv, blv):
    raise NotImplementedError("write your pallas kernel here")


# speedup vs baseline: 17.8156x; 17.8156x over previous
import functools

import jax
import jax.numpy as jnp
from jax import lax
from jax.experimental import pallas as pl
from jax.experimental.pallas import tpu as pltpu
from jax.experimental.pallas import tpu_sc as plsc

N_NODES = 10000
N_EDGES = 320000
IN_DIM = 128
HIDDEN_DIM = 128
Z_DIM = 64

BATCH = 128
NB = N_EDGES // BATCH
NC, NS = 2, 16
NB_PER_CORE = NB // NC
NB_PER_TILE = NB_PER_CORE // NS
NB_REM = NB_PER_CORE % NS
STRIPE = 624
STRIPE_REM = N_NODES - NS * STRIPE

DEG_W = 128


def _make_sc_pass(D, gather):
    mesh = plsc.VectorSubcoreMesh(core_axis_name="c", subcore_axis_name="s")

    scratch = [
        pltpu.VMEM((BATCH,), jnp.int32),
        pltpu.VMEM((BATCH,), jnp.int32),
        pltpu.VMEM((BATCH, D), jnp.float32),
        pltpu.VMEM_SHARED((N_NODES, D), jnp.float32),
        pltpu.SemaphoreType.DMA,
    ]

    @functools.partial(
        pl.kernel,
        mesh=mesh,
        out_type=jax.ShapeDtypeStruct((NC, N_NODES, D), jnp.float32),
        scratch_types=scratch,
    )
    def k(table, src1d, dst1d, zeros_hbm, out, sidx, didx, rows, acc, sem):
        c = lax.axis_index("c")
        s = lax.axis_index("s")
        r0 = s * STRIPE
        pltpu.sync_copy(zeros_hbm.at[pl.ds(r0, STRIPE)],
                        acc.at[pl.ds(r0, STRIPE)])

        @pl.when(s == NS - 1)
        def _():
            pltpu.sync_copy(zeros_hbm.at[pl.ds(NS * STRIPE, STRIPE_REM)],
                            acc.at[pl.ds(NS * STRIPE, STRIPE_REM)])

        if not gather:
            pltpu.sync_copy(table, rows)
        plsc.subcore_barrier()

        start = c * NB_PER_CORE + s * NB_PER_TILE + jnp.minimum(s, NB_REM)
        n = NB_PER_TILE + jnp.where(s < NB_REM, 1, 0)

        def body(i, carry):
            j = start + i
            pltpu.sync_copy(dst1d.at[pl.ds(j * BATCH, BATCH)], didx)
            if gather:
                pltpu.sync_copy(src1d.at[pl.ds(j * BATCH, BATCH)], sidx)
                pltpu.async_copy(table.at[sidx], rows, sem).wait()
            pltpu.sync_copy(rows, acc.at[didx], add=True)
            return carry

        lax.fori_loop(0, n, body, 0)
        plsc.subcore_barrier()
        pltpu.sync_copy(acc.at[pl.ds(r0, STRIPE)],
                        out.at[c].at[pl.ds(r0, STRIPE)])

        @pl.when(s == NS - 1)
        def _():
            pltpu.sync_copy(acc.at[pl.ds(NS * STRIPE, STRIPE_REM)],
                            out.at[c].at[pl.ds(NS * STRIPE, STRIPE_REM)])

    return k


_sc_deg = _make_sc_pass(DEG_W, gather=False)
_sc_spmm = _make_sc_pass(HIDDEN_DIM, gather=True)

R = 1000
GRID = N_NODES // R


def _dinv_of(degp_ref):
    deg = degp_ref[0] + degp_ref[1] + 1.0
    return lax.rsqrt(deg)[:, 0:1]


def _tc1_body(x_ref, w_ref, degp_ref, o_ref):
    h0 = jnp.dot(x_ref[...], w_ref[...], preferred_element_type=jnp.float32)
    o_ref[...] = h0 * _dinv_of(degp_ref)


def _tc2_body(s_ref, h_ref, degp_ref, b_ref, o_ref):
    dinv = _dinv_of(degp_ref)
    h1 = jnp.maximum(dinv * (s_ref[0] + s_ref[1] + h_ref[...]) + b_ref[...], 0.0)
    o_ref[...] = dinv * h1


def _tc3_body(s_ref, h_ref, degp_ref, wm_ref, bm_ref, wl_ref, bl_ref,
              mu_ref, lv_ref):
    dinv = _dinv_of(degp_ref)
    g = dinv * (s_ref[0] + s_ref[1] + h_ref[...])
    mu_ref[...] = jnp.dot(g, wm_ref[...], preferred_element_type=jnp.float32) + bm_ref[...]
    lv_ref[...] = jnp.dot(g, wl_ref[...], preferred_element_type=jnp.float32) + bl_ref[...]


def _row_spec(d):
    return pl.BlockSpec((R, d), lambda i: (i, 0))


def _part_spec(d):
    return pl.BlockSpec((NC, R, d), lambda i: (0, i, 0))


def _full_spec(a, b):
    return pl.BlockSpec((a, b), lambda i: (0, 0))


_tc1 = pl.pallas_call(
    _tc1_body,
    grid=(GRID,),
    in_specs=[_row_spec(IN_DIM), _full_spec(IN_DIM, HIDDEN_DIM), _part_spec(DEG_W)],
    out_specs=_row_spec(HIDDEN_DIM),
    out_shape=jax.ShapeDtypeStruct((N_NODES, HIDDEN_DIM), jnp.float32),
)

_tc2 = pl.pallas_call(
    _tc2_body,
    grid=(GRID,),
    in_specs=[_part_spec(HIDDEN_DIM), _row_spec(HIDDEN_DIM), _part_spec(DEG_W),
              _full_spec(1, HIDDEN_DIM)],
    out_specs=_row_spec(HIDDEN_DIM),
    out_shape=jax.ShapeDtypeStruct((N_NODES, HIDDEN_DIM), jnp.float32),
)

_tc3 = pl.pallas_call(
    _tc3_body,
    grid=(GRID,),
    in_specs=[_part_spec(HIDDEN_DIM), _row_spec(HIDDEN_DIM), _part_spec(DEG_W),
              _full_spec(HIDDEN_DIM, Z_DIM), _full_spec(1, Z_DIM),
              _full_spec(HIDDEN_DIM, Z_DIM), _full_spec(1, Z_DIM)],
    out_specs=[_row_spec(Z_DIM), _row_spec(Z_DIM)],
    out_shape=[jax.ShapeDtypeStruct((N_NODES, Z_DIM), jnp.float32),
               jax.ShapeDtypeStruct((N_NODES, Z_DIM), jnp.float32)],
)


def kernel(x, edge_index, W1, b1, Wmu, bmu, Wlv, blv):
    src1d = edge_index[0].astype(jnp.int32).reshape(N_EDGES)
    dst1d = edge_index[1].astype(jnp.int32).reshape(N_EDGES)
    ones_rows = jnp.ones((BATCH, DEG_W), jnp.float32)
    z_deg = jnp.zeros((N_NODES, DEG_W), jnp.float32)
    z_h = jnp.zeros((N_NODES, HIDDEN_DIM), jnp.float32)

    degp = _sc_deg(ones_rows, src1d, dst1d, z_deg)
    h0s = _tc1(x, W1, degp)
    s1 = _sc_spmm(h0s, src1d, dst1d, z_h)
    h1s = _tc2(s1, h0s, degp, b1.reshape(1, HIDDEN_DIM))
    s2 = _sc_spmm(h1s, src1d, dst1d, z_h)
    mu, lv = _tc3(s2, h1s, degp, Wmu, bmu.reshape(1, Z_DIM),
                  Wlv, blv.reshape(1, Z_DIM))
    return (mu, lv)


# speedup vs baseline: 33.1028x; 1.8581x over previous
import functools

import jax
import jax.numpy as jnp
from jax import lax
from jax.experimental import pallas as pl
from jax.experimental.pallas import tpu as pltpu
from jax.experimental.pallas import tpu_sc as plsc

N_NODES = 10000
N_EDGES = 320000
IN_DIM = 128
HIDDEN_DIM = 128
Z_DIM = 64

B = 128
NC, NS = 2, 16
NW = NC * NS
EPT = N_EDGES // NW
NBT = EPT // B
TAIL = EPT - NBT * B
CH = 3
CBT = NBT // CH
CPAIRS = CBT // 2
CW = CBT * B
STRIPE = 624
STRIPE_REM = N_NODES - NS * STRIPE

_mesh = plsc.VectorSubcoreMesh(core_axis_name="c", subcore_axis_name="s")


@functools.partial(
    pl.kernel,
    mesh=_mesh,
    compiler_params=pltpu.CompilerParams(needs_layout_passes=False),
    out_type=jax.ShapeDtypeStruct((NW * N_NODES,), jnp.float32),
    scratch_types=[
        pltpu.VMEM((EPT,), jnp.int32),
        pltpu.VMEM((N_NODES,), jnp.float32),
    ],
)
def _sc_deg(dst1d, zeros_hbm, out, dvm, tabv):
    tid = lax.axis_index("c") * NS + lax.axis_index("s")
    pltpu.sync_copy(dst1d.at[pl.ds(tid * EPT, EPT)], dvm)
    pltpu.sync_copy(zeros_hbm, tabv)
    ones = jnp.ones((16,), jnp.float32)

    def body(i, carry):
        for u in range(4):
            iv = dvm[pl.ds((i * 4 + u) * 16, 16)]
            plsc.addupdate_scatter(tabv, [iv], ones)
        return carry

    lax.fori_loop(0, EPT // 64, body, 0)
    iv = dvm[pl.ds(EPT - 16, 16)]
    plsc.addupdate_scatter(tabv, [iv], ones)
    pltpu.sync_copy(tabv, out.at[pl.ds(tid * N_NODES, N_NODES)])


@functools.partial(
    pl.kernel,
    mesh=_mesh,
    out_type=jax.ShapeDtypeStruct((NC, N_NODES, HIDDEN_DIM), jnp.float32),
    scratch_types=[
        pltpu.VMEM((CW,), jnp.int32),
        pltpu.VMEM((CW,), jnp.int32),
        pltpu.VMEM((B, HIDDEN_DIM), jnp.float32),
        pltpu.VMEM((B, HIDDEN_DIM), jnp.float32),
        pltpu.VMEM_SHARED((N_NODES, HIDDEN_DIM), jnp.float32),
        pltpu.SemaphoreType.DMA,
        pltpu.SemaphoreType.DMA,
        pltpu.SemaphoreType.DMA,
        pltpu.SemaphoreType.DMA,
    ],
)
def _sc_spmm(table, src1d, dst1d, zeros_hbm, out,
             svm, dvm, rowsA, rowsB, acc, gA, gB, sA, sB):
    c = lax.axis_index("c")
    s = lax.axis_index("s")
    r0 = s * STRIPE
    pltpu.sync_copy(zeros_hbm.at[pl.ds(r0, STRIPE)], acc.at[pl.ds(r0, STRIPE)])

    @pl.when(s == NS - 1)
    def _():
        pltpu.sync_copy(zeros_hbm.at[pl.ds(NS * STRIPE, STRIPE_REM)],
                        acc.at[pl.ds(NS * STRIPE, STRIPE_REM)])

    w = (c * NS + s) * EPT
    plsc.subcore_barrier()

    def sv(j):
        return svm.at[pl.ds(j * B, B)]

    def dv(j):
        return dvm.at[pl.ds(j * B, B)]

    def chunk(ch, carry):
        wch = w + ch * CW
        pltpu.sync_copy(src1d.at[pl.ds(wch, CW)], svm)
        pltpu.sync_copy(dst1d.at[pl.ds(wch, CW)], dvm)
        pltpu.async_copy(table.at[sv(0)], rowsA, gA)

        def body(i, carry):
            jA = 2 * i
            jB = 2 * i + 1
            pltpu.make_async_copy(table.at[sv(jA)], rowsA, gA).wait()
            pltpu.async_copy(rowsA, acc.at[dv(jA)], sA, add=True)

            @pl.when(i > 0)
            def _():
                pltpu.make_async_copy(rowsB, acc.at[dv(jB)], sB).wait()

            pltpu.async_copy(table.at[sv(jB)], rowsB, gB)
            pltpu.make_async_copy(table.at[sv(jB)], rowsB, gB).wait()
            pltpu.async_copy(rowsB, acc.at[dv(jB)], sB, add=True)
            pltpu.make_async_copy(rowsA, acc.at[dv(jA)], sA).wait()

            @pl.when(i < CPAIRS - 1)
            def _():
                pltpu.async_copy(table.at[sv(jA + 2)], rowsA, gA)
            return carry

        lax.fori_loop(0, CPAIRS, body, 0)
        pltpu.make_async_copy(rowsB, acc.at[dv(CBT - 1)], sB).wait()
        return carry

    lax.fori_loop(0, CH, chunk, 0)

    pltpu.sync_copy(src1d.at[pl.ds(w + NBT * B, TAIL)], svm.at[pl.ds(0, TAIL)])
    pltpu.sync_copy(dst1d.at[pl.ds(w + NBT * B, TAIL)], dvm.at[pl.ds(0, TAIL)])
    rt = rowsA.at[pl.ds(0, TAIL)]
    pltpu.async_copy(table.at[svm.at[pl.ds(0, TAIL)]], rt, gA).wait()
    pltpu.sync_copy(rt, acc.at[dvm.at[pl.ds(0, TAIL)]], add=True)

    plsc.subcore_barrier()
    pltpu.sync_copy(acc.at[pl.ds(r0, STRIPE)], out.at[c].at[pl.ds(r0, STRIPE)])

    @pl.when(s == NS - 1)
    def _():
        pltpu.sync_copy(acc.at[pl.ds(NS * STRIPE, STRIPE_REM)],
                        out.at[c].at[pl.ds(NS * STRIPE, STRIPE_REM)])


R = 1000
GRID = N_NODES // R
def _dinv_of(degp_ref):
    deg = jnp.sum(degp_ref[...], axis=1, keepdims=True)
    return lax.rsqrt(deg + 1.0)


def _tc1_body(x_ref, w_ref, degp_ref, o_ref):
    h0 = jnp.dot(x_ref[...], w_ref[...], preferred_element_type=jnp.float32)
    o_ref[...] = h0 * _dinv_of(degp_ref)


def _tc2_body(s_ref, h_ref, degp_ref, b_ref, o_ref):
    dinv = _dinv_of(degp_ref)
    h1 = jnp.maximum(dinv * (s_ref[0] + s_ref[1] + h_ref[...]) + b_ref[...], 0.0)
    o_ref[...] = dinv * h1


def _tc3_body(s_ref, h_ref, degp_ref, wm_ref, bm_ref, wl_ref, bl_ref,
              mu_ref, lv_ref):
    g = _dinv_of(degp_ref) * (s_ref[0] + s_ref[1] + h_ref[...])
    mu_ref[...] = jnp.dot(g, wm_ref[...], preferred_element_type=jnp.float32) + bm_ref[...]
    lv_ref[...] = jnp.dot(g, wl_ref[...], preferred_element_type=jnp.float32) + bl_ref[...]


def _row_spec(d):
    return pl.BlockSpec((R, d), lambda i: (i, 0))


def _part_spec(d):
    return pl.BlockSpec((NC, R, d), lambda i: (0, i, 0))


_deg_spec = pl.BlockSpec((R, NW), lambda i: (i, 0))


def _full_spec(a, b):
    return pl.BlockSpec((a, b), lambda i: (0, 0))


_tc1 = pl.pallas_call(
    _tc1_body,
    grid=(GRID,),
    in_specs=[_row_spec(IN_DIM), _full_spec(IN_DIM, HIDDEN_DIM), _deg_spec],
    out_specs=_row_spec(HIDDEN_DIM),
    out_shape=jax.ShapeDtypeStruct((N_NODES, HIDDEN_DIM), jnp.float32),
)

_tc2 = pl.pallas_call(
    _tc2_body,
    grid=(GRID,),
    in_specs=[_part_spec(HIDDEN_DIM), _row_spec(HIDDEN_DIM), _deg_spec,
              _full_spec(1, HIDDEN_DIM)],
    out_specs=_row_spec(HIDDEN_DIM),
    out_shape=jax.ShapeDtypeStruct((N_NODES, HIDDEN_DIM), jnp.float32),
)

_tc3 = pl.pallas_call(
    _tc3_body,
    grid=(GRID,),
    in_specs=[_part_spec(HIDDEN_DIM), _row_spec(HIDDEN_DIM), _deg_spec,
              _full_spec(HIDDEN_DIM, Z_DIM), _full_spec(1, Z_DIM),
              _full_spec(HIDDEN_DIM, Z_DIM), _full_spec(1, Z_DIM)],
    out_specs=[_row_spec(Z_DIM), _row_spec(Z_DIM)],
    out_shape=[jax.ShapeDtypeStruct((N_NODES, Z_DIM), jnp.float32),
               jax.ShapeDtypeStruct((N_NODES, Z_DIM), jnp.float32)],
)


def kernel(x, edge_index, W1, b1, Wmu, bmu, Wlv, blv):
    src1d = edge_index[0].astype(jnp.int32).reshape(N_EDGES)
    dst1d = edge_index[1].astype(jnp.int32).reshape(N_EDGES)
    z1 = jnp.zeros((N_NODES,), jnp.float32)
    z_h = jnp.zeros((N_NODES, HIDDEN_DIM), jnp.float32)

    degp = _sc_deg(dst1d, z1).reshape(NW, N_NODES).T
    h0s = _tc1(x, W1, degp)
    s1 = _sc_spmm(h0s, src1d, dst1d, z_h)
    h1s = _tc2(s1, h0s, degp, b1.reshape(1, HIDDEN_DIM))
    s2 = _sc_spmm(h1s, src1d, dst1d, z_h)
    mu, lv = _tc3(s2, h1s, degp, Wmu, bmu.reshape(1, Z_DIM),
                  Wlv, blv.reshape(1, Z_DIM))
    return (mu, lv)


# speedup vs baseline: 33.1552x; 1.0016x over previous
import functools

import jax
import jax.numpy as jnp
from jax import lax
from jax.experimental import pallas as pl
from jax.experimental.pallas import tpu as pltpu
from jax.experimental.pallas import tpu_sc as plsc

N_NODES = 10000
N_EDGES = 320000
IN_DIM = 128
HIDDEN_DIM = 128
Z_DIM = 64

B = 128
NC, NS = 2, 16
NW = NC * NS
EPT = N_EDGES // NW
NBT = EPT // B
TAIL = EPT - NBT * B
CH = 3
CBT = NBT // CH
CPAIRS = CBT // 2
CW = CBT * B
STRIPE = 624
STRIPE_REM = N_NODES - NS * STRIPE

_mesh = plsc.VectorSubcoreMesh(core_axis_name="c", subcore_axis_name="s")


@functools.partial(
    pl.kernel,
    mesh=_mesh,
    compiler_params=pltpu.CompilerParams(needs_layout_passes=False),
    out_type=jax.ShapeDtypeStruct((NW * N_NODES,), jnp.float32),
    scratch_types=[
        pltpu.VMEM((EPT,), jnp.int32),
        pltpu.VMEM((N_NODES,), jnp.float32),
    ],
)
def _sc_deg(dst1d, zeros_hbm, out, dvm, tabv):
    tid = lax.axis_index("c") * NS + lax.axis_index("s")
    pltpu.sync_copy(dst1d.at[pl.ds(tid * EPT, EPT)], dvm)
    pltpu.sync_copy(zeros_hbm, tabv)
    ones = jnp.ones((16,), jnp.float32)

    def body(i, carry):
        for u in range(4):
            iv = dvm[pl.ds((i * 4 + u) * 16, 16)]
            plsc.addupdate_scatter(tabv, [iv], ones)
        return carry

    lax.fori_loop(0, EPT // 64, body, 0)
    iv = dvm[pl.ds(EPT - 16, 16)]
    plsc.addupdate_scatter(tabv, [iv], ones)
    pltpu.sync_copy(tabv, out.at[pl.ds(tid * N_NODES, N_NODES)])


@functools.partial(
    pl.kernel,
    mesh=_mesh,
    out_type=jax.ShapeDtypeStruct((NC, N_NODES, HIDDEN_DIM), jnp.float32),
    scratch_types=[
        pltpu.VMEM((CW,), jnp.int32),
        pltpu.VMEM((CW,), jnp.int32),
        pltpu.VMEM((B, HIDDEN_DIM), jnp.float32),
        pltpu.VMEM((B, HIDDEN_DIM), jnp.float32),
        pltpu.VMEM_SHARED((N_NODES, HIDDEN_DIM), jnp.float32),
        pltpu.SemaphoreType.DMA,
        pltpu.SemaphoreType.DMA,
        pltpu.SemaphoreType.DMA,
        pltpu.SemaphoreType.DMA,
    ],
)
def _sc_spmm(table, src1d, dst1d, zeros_hbm, out,
             svm, dvm, rowsA, rowsB, acc, gA, gB, sA, sB):
    c = lax.axis_index("c")
    s = lax.axis_index("s")
    r0 = s * STRIPE
    pltpu.sync_copy(zeros_hbm.at[pl.ds(r0, STRIPE)], acc.at[pl.ds(r0, STRIPE)])

    @pl.when(s == NS - 1)
    def _():
        pltpu.sync_copy(zeros_hbm.at[pl.ds(NS * STRIPE, STRIPE_REM)],
                        acc.at[pl.ds(NS * STRIPE, STRIPE_REM)])

    w = (c * NS + s) * EPT
    plsc.subcore_barrier()

    def sv(j):
        return svm.at[pl.ds(j * B, B)]

    def dv(j):
        return dvm.at[pl.ds(j * B, B)]

    def chunk(ch, carry):
        wch = w + ch * CW
        pltpu.sync_copy(src1d.at[pl.ds(wch, CW)], svm)
        pltpu.sync_copy(dst1d.at[pl.ds(wch, CW)], dvm)
        pltpu.async_copy(table.at[sv(0)], rowsA, gA)

        def body(i, carry):
            jA = 2 * i
            jB = 2 * i + 1
            pltpu.make_async_copy(table.at[sv(jA)], rowsA, gA).wait()

            @pl.when(i > 0)
            def _():
                pltpu.make_async_copy(rowsB, acc.at[dv(jB)], sB).wait()

            descGB = pltpu.async_copy(table.at[sv(jB)], rowsB, gB)
            descSA = pltpu.async_copy(rowsA, acc.at[dv(jA)], sA, add=True)
            descGB.wait()
            pltpu.async_copy(rowsB, acc.at[dv(jB)], sB, add=True)
            descSA.wait()

            @pl.when(i < CPAIRS - 1)
            def _():
                pltpu.async_copy(table.at[sv(jA + 2)], rowsA, gA)
            return carry

        lax.fori_loop(0, CPAIRS, body, 0)
        pltpu.make_async_copy(rowsB, acc.at[dv(CBT - 1)], sB).wait()
        return carry

    lax.fori_loop(0, CH, chunk, 0)

    pltpu.sync_copy(src1d.at[pl.ds(w + NBT * B, TAIL)], svm.at[pl.ds(0, TAIL)])
    pltpu.sync_copy(dst1d.at[pl.ds(w + NBT * B, TAIL)], dvm.at[pl.ds(0, TAIL)])
    rt = rowsA.at[pl.ds(0, TAIL)]
    pltpu.async_copy(table.at[svm.at[pl.ds(0, TAIL)]], rt, gA).wait()
    pltpu.sync_copy(rt, acc.at[dvm.at[pl.ds(0, TAIL)]], add=True)

    plsc.subcore_barrier()
    pltpu.sync_copy(acc.at[pl.ds(r0, STRIPE)], out.at[c].at[pl.ds(r0, STRIPE)])

    @pl.when(s == NS - 1)
    def _():
        pltpu.sync_copy(acc.at[pl.ds(NS * STRIPE, STRIPE_REM)],
                        out.at[c].at[pl.ds(NS * STRIPE, STRIPE_REM)])


R = 1000
GRID = N_NODES // R
def _dinv_of(degp_ref):
    deg = jnp.sum(degp_ref[...], axis=1, keepdims=True)
    return lax.rsqrt(deg + 1.0)


def _tc1_body(x_ref, w_ref, degp_ref, o_ref):
    h0 = jnp.dot(x_ref[...], w_ref[...], preferred_element_type=jnp.float32)
    o_ref[...] = h0 * _dinv_of(degp_ref)


def _tc2_body(s_ref, h_ref, degp_ref, b_ref, o_ref):
    dinv = _dinv_of(degp_ref)
    h1 = jnp.maximum(dinv * (s_ref[0] + s_ref[1] + h_ref[...]) + b_ref[...], 0.0)
    o_ref[...] = dinv * h1


def _tc3_body(s_ref, h_ref, degp_ref, wm_ref, bm_ref, wl_ref, bl_ref,
              mu_ref, lv_ref):
    g = _dinv_of(degp_ref) * (s_ref[0] + s_ref[1] + h_ref[...])
    mu_ref[...] = jnp.dot(g, wm_ref[...], preferred_element_type=jnp.float32) + bm_ref[...]
    lv_ref[...] = jnp.dot(g, wl_ref[...], preferred_element_type=jnp.float32) + bl_ref[...]


def _row_spec(d):
    return pl.BlockSpec((R, d), lambda i: (i, 0))


def _part_spec(d):
    return pl.BlockSpec((NC, R, d), lambda i: (0, i, 0))


_deg_spec = pl.BlockSpec((R, NW), lambda i: (i, 0))


def _full_spec(a, b):
    return pl.BlockSpec((a, b), lambda i: (0, 0))


_tc1 = pl.pallas_call(
    _tc1_body,
    grid=(GRID,),
    in_specs=[_row_spec(IN_DIM), _full_spec(IN_DIM, HIDDEN_DIM), _deg_spec],
    out_specs=_row_spec(HIDDEN_DIM),
    out_shape=jax.ShapeDtypeStruct((N_NODES, HIDDEN_DIM), jnp.float32),
)

_tc2 = pl.pallas_call(
    _tc2_body,
    grid=(GRID,),
    in_specs=[_part_spec(HIDDEN_DIM), _row_spec(HIDDEN_DIM), _deg_spec,
              _full_spec(1, HIDDEN_DIM)],
    out_specs=_row_spec(HIDDEN_DIM),
    out_shape=jax.ShapeDtypeStruct((N_NODES, HIDDEN_DIM), jnp.float32),
)

_tc3 = pl.pallas_call(
    _tc3_body,
    grid=(GRID,),
    in_specs=[_part_spec(HIDDEN_DIM), _row_spec(HIDDEN_DIM), _deg_spec,
              _full_spec(HIDDEN_DIM, Z_DIM), _full_spec(1, Z_DIM),
              _full_spec(HIDDEN_DIM, Z_DIM), _full_spec(1, Z_DIM)],
    out_specs=[_row_spec(Z_DIM), _row_spec(Z_DIM)],
    out_shape=[jax.ShapeDtypeStruct((N_NODES, Z_DIM), jnp.float32),
               jax.ShapeDtypeStruct((N_NODES, Z_DIM), jnp.float32)],
)


def kernel(x, edge_index, W1, b1, Wmu, bmu, Wlv, blv):
    src1d = edge_index[0].astype(jnp.int32).reshape(N_EDGES)
    dst1d = edge_index[1].astype(jnp.int32).reshape(N_EDGES)
    z1 = jnp.zeros((N_NODES,), jnp.float32)
    z_h = jnp.zeros((N_NODES, HIDDEN_DIM), jnp.float32)

    degp = _sc_deg(dst1d, z1).reshape(NW, N_NODES).T
    h0s = _tc1(x, W1, degp)
    s1 = _sc_spmm(h0s, src1d, dst1d, z_h)
    h1s = _tc2(s1, h0s, degp, b1.reshape(1, HIDDEN_DIM))
    s2 = _sc_spmm(h1s, src1d, dst1d, z_h)
    mu, lv = _tc3(s2, h1s, degp, Wmu, bmu.reshape(1, Z_DIM),
                  Wlv, blv.reshape(1, Z_DIM))
    return (mu, lv)


# speedup vs baseline: 33.2801x; 1.0038x over previous
import functools

import jax
import jax.numpy as jnp
from jax import lax
from jax.experimental import pallas as pl
from jax.experimental.pallas import tpu as pltpu
from jax.experimental.pallas import tpu_sc as plsc

N_NODES = 10000
N_EDGES = 320000
IN_DIM = 128
HIDDEN_DIM = 128
Z_DIM = 64

B = 128
NC, NS = 2, 16
NW = NC * NS
EPT = N_EDGES // NW
NBT = EPT // B
TAIL = EPT - NBT * B
CH = 3
CBT = NBT // CH
CPAIRS = CBT // 2
CW = CBT * B
STRIPE = 624
STRIPE_REM = N_NODES - NS * STRIPE

_mesh = plsc.VectorSubcoreMesh(core_axis_name="c", subcore_axis_name="s")


@functools.partial(
    pl.kernel,
    mesh=_mesh,
    compiler_params=pltpu.CompilerParams(needs_layout_passes=False),
    out_type=jax.ShapeDtypeStruct((NW * N_NODES,), jnp.float32),
    scratch_types=[
        pltpu.VMEM((EPT,), jnp.int32),
        pltpu.VMEM((N_NODES,), jnp.float32),
    ],
)
def _sc_deg(dst1d, zeros_hbm, out, dvm, tabv):
    tid = lax.axis_index("c") * NS + lax.axis_index("s")
    pltpu.sync_copy(dst1d.at[pl.ds(tid * EPT, EPT)], dvm)
    pltpu.sync_copy(zeros_hbm, tabv)
    ones = jnp.ones((16,), jnp.float32)

    def body(i, carry):
        for u in range(4):
            iv = dvm[pl.ds((i * 4 + u) * 16, 16)]
            plsc.addupdate_scatter(tabv, [iv], ones)
        return carry

    lax.fori_loop(0, EPT // 64, body, 0)
    iv = dvm[pl.ds(EPT - 16, 16)]
    plsc.addupdate_scatter(tabv, [iv], ones)
    for r in range(N_NODES // 1000):
        pltpu.sync_copy(tabv.at[pl.ds(r * 1000, 1000)],
                        out.at[pl.ds(r * NW * 1000 + tid * 1000, 1000)])


@functools.partial(
    pl.kernel,
    mesh=_mesh,
    out_type=jax.ShapeDtypeStruct((NC, N_NODES, HIDDEN_DIM), jnp.float32),
    scratch_types=[
        pltpu.VMEM((CW,), jnp.int32),
        pltpu.VMEM((CW,), jnp.int32),
        pltpu.VMEM((B, HIDDEN_DIM), jnp.float32),
        pltpu.VMEM((B, HIDDEN_DIM), jnp.float32),
        pltpu.VMEM_SHARED((N_NODES, HIDDEN_DIM), jnp.float32),
        pltpu.SemaphoreType.DMA,
        pltpu.SemaphoreType.DMA,
        pltpu.SemaphoreType.DMA,
        pltpu.SemaphoreType.DMA,
    ],
)
def _sc_spmm(table, src1d, dst1d, zeros_hbm, out,
             svm, dvm, rowsA, rowsB, acc, gA, gB, sA, sB):
    c = lax.axis_index("c")
    s = lax.axis_index("s")
    r0 = s * STRIPE
    pltpu.sync_copy(zeros_hbm.at[pl.ds(r0, STRIPE)], acc.at[pl.ds(r0, STRIPE)])

    @pl.when(s == NS - 1)
    def _():
        pltpu.sync_copy(zeros_hbm.at[pl.ds(NS * STRIPE, STRIPE_REM)],
                        acc.at[pl.ds(NS * STRIPE, STRIPE_REM)])

    w = (c * NS + s) * EPT
    plsc.subcore_barrier()

    def sv(j):
        return svm.at[pl.ds(j * B, B)]

    def dv(j):
        return dvm.at[pl.ds(j * B, B)]

    def chunk(ch, carry):
        wch = w + ch * CW
        pltpu.sync_copy(src1d.at[pl.ds(wch, CW)], svm)
        pltpu.sync_copy(dst1d.at[pl.ds(wch, CW)], dvm)
        pltpu.async_copy(table.at[sv(0)], rowsA, gA)

        def body(i, carry):
            jA = 2 * i
            jB = 2 * i + 1
            pltpu.make_async_copy(table.at[sv(jA)], rowsA, gA).wait()

            @pl.when(i > 0)
            def _():
                pltpu.make_async_copy(rowsB, acc.at[dv(jB)], sB).wait()

            descGB = pltpu.async_copy(table.at[sv(jB)], rowsB, gB)
            descSA = pltpu.async_copy(rowsA, acc.at[dv(jA)], sA, add=True)
            descGB.wait()
            pltpu.async_copy(rowsB, acc.at[dv(jB)], sB, add=True)
            descSA.wait()

            @pl.when(i < CPAIRS - 1)
            def _():
                pltpu.async_copy(table.at[sv(jA + 2)], rowsA, gA)
            return carry

        lax.fori_loop(0, CPAIRS, body, 0)
        pltpu.make_async_copy(rowsB, acc.at[dv(CBT - 1)], sB).wait()
        return carry

    lax.fori_loop(0, CH, chunk, 0)

    pltpu.sync_copy(src1d.at[pl.ds(w + NBT * B, TAIL)], svm.at[pl.ds(0, TAIL)])
    pltpu.sync_copy(dst1d.at[pl.ds(w + NBT * B, TAIL)], dvm.at[pl.ds(0, TAIL)])
    rt = rowsA.at[pl.ds(0, TAIL)]
    pltpu.async_copy(table.at[svm.at[pl.ds(0, TAIL)]], rt, gA).wait()
    pltpu.sync_copy(rt, acc.at[dvm.at[pl.ds(0, TAIL)]], add=True)

    plsc.subcore_barrier()
    pltpu.sync_copy(acc.at[pl.ds(r0, STRIPE)], out.at[c].at[pl.ds(r0, STRIPE)])

    @pl.when(s == NS - 1)
    def _():
        pltpu.sync_copy(acc.at[pl.ds(NS * STRIPE, STRIPE_REM)],
                        out.at[c].at[pl.ds(NS * STRIPE, STRIPE_REM)])


R = 1000
GRID = N_NODES // R
def _dinv_of(degp_ref):
    ones = jnp.ones((NW, HIDDEN_DIM), jnp.float32)
    deg = lax.dot_general(degp_ref[0], ones, (((0,), (0,)), ((), ())),
                          preferred_element_type=jnp.float32)
    return lax.rsqrt(deg + 1.0)


def _tc1_body(x_ref, w_ref, degp_ref, o_ref):
    h0 = jnp.dot(x_ref[...], w_ref[...], preferred_element_type=jnp.float32)
    o_ref[...] = h0 * _dinv_of(degp_ref)


def _tc2_body(s_ref, h_ref, degp_ref, b_ref, o_ref):
    dinv = _dinv_of(degp_ref)
    h1 = jnp.maximum(dinv * (s_ref[0] + s_ref[1] + h_ref[...]) + b_ref[...], 0.0)
    o_ref[...] = dinv * h1


def _tc3_body(s_ref, h_ref, degp_ref, wm_ref, bm_ref, wl_ref, bl_ref,
              mu_ref, lv_ref):
    g = _dinv_of(degp_ref) * (s_ref[0] + s_ref[1] + h_ref[...])
    mu_ref[...] = jnp.dot(g, wm_ref[...], preferred_element_type=jnp.float32) + bm_ref[...]
    lv_ref[...] = jnp.dot(g, wl_ref[...], preferred_element_type=jnp.float32) + bl_ref[...]


def _row_spec(d):
    return pl.BlockSpec((R, d), lambda i: (i, 0))


def _part_spec(d):
    return pl.BlockSpec((NC, R, d), lambda i: (0, i, 0))


_deg_spec = pl.BlockSpec((1, NW, R), lambda i: (i, 0, 0))


def _full_spec(a, b):
    return pl.BlockSpec((a, b), lambda i: (0, 0))


_tc1 = pl.pallas_call(
    _tc1_body,
    grid=(GRID,),
    in_specs=[_row_spec(IN_DIM), _full_spec(IN_DIM, HIDDEN_DIM), _deg_spec],
    out_specs=_row_spec(HIDDEN_DIM),
    out_shape=jax.ShapeDtypeStruct((N_NODES, HIDDEN_DIM), jnp.float32),
)

_tc2 = pl.pallas_call(
    _tc2_body,
    grid=(GRID,),
    in_specs=[_part_spec(HIDDEN_DIM), _row_spec(HIDDEN_DIM), _deg_spec,
              _full_spec(1, HIDDEN_DIM)],
    out_specs=_row_spec(HIDDEN_DIM),
    out_shape=jax.ShapeDtypeStruct((N_NODES, HIDDEN_DIM), jnp.float32),
)

_tc3 = pl.pallas_call(
    _tc3_body,
    grid=(GRID,),
    in_specs=[_part_spec(HIDDEN_DIM), _row_spec(HIDDEN_DIM), _deg_spec,
              _full_spec(HIDDEN_DIM, Z_DIM), _full_spec(1, Z_DIM),
              _full_spec(HIDDEN_DIM, Z_DIM), _full_spec(1, Z_DIM)],
    out_specs=[_row_spec(Z_DIM), _row_spec(Z_DIM)],
    out_shape=[jax.ShapeDtypeStruct((N_NODES, Z_DIM), jnp.float32),
               jax.ShapeDtypeStruct((N_NODES, Z_DIM), jnp.float32)],
)


def kernel(x, edge_index, W1, b1, Wmu, bmu, Wlv, blv):
    src1d = edge_index[0].astype(jnp.int32).reshape(N_EDGES)
    dst1d = edge_index[1].astype(jnp.int32).reshape(N_EDGES)
    z1 = jnp.zeros((N_NODES,), jnp.float32)
    z_h = jnp.zeros((N_NODES, HIDDEN_DIM), jnp.float32)

    degp = _sc_deg(dst1d, z1).reshape(GRID, NW, R)
    h0s = _tc1(x, W1, degp)
    s1 = _sc_spmm(h0s, src1d, dst1d, z_h)
    h1s = _tc2(s1, h0s, degp, b1.reshape(1, HIDDEN_DIM))
    s2 = _sc_spmm(h1s, src1d, dst1d, z_h)
    mu, lv = _tc3(s2, h1s, degp, Wmu, bmu.reshape(1, Z_DIM),
                  Wlv, blv.reshape(1, Z_DIM))
    return (mu, lv)


# speedup vs baseline: 33.4389x; 1.0048x over previous
import functools

import jax
import jax.numpy as jnp
from jax import lax
from jax.experimental import pallas as pl
from jax.experimental.pallas import tpu as pltpu
from jax.experimental.pallas import tpu_sc as plsc

N_NODES = 10000
N_EDGES = 320000
IN_DIM = 128
HIDDEN_DIM = 128
Z_DIM = 64

B = 128
NC, NS = 2, 16
NW = NC * NS
EPT = N_EDGES // NW
NBT = EPT // B
TAIL = EPT - NBT * B
CH = 3
CBT = NBT // CH
CPAIRS = CBT // 2
CW = CBT * B
STRIPE = 624
STRIPE_REM = N_NODES - NS * STRIPE

_mesh = plsc.VectorSubcoreMesh(core_axis_name="c", subcore_axis_name="s")


@functools.partial(
    pl.kernel,
    mesh=_mesh,
    compiler_params=pltpu.CompilerParams(needs_layout_passes=False),
    out_type=jax.ShapeDtypeStruct((NW * N_NODES,), jnp.float32),
    scratch_types=[
        pltpu.VMEM((EPT,), jnp.int32),
        pltpu.VMEM((N_NODES,), jnp.float32),
    ],
)
def _sc_deg(dst1d, zeros_hbm, out, dvm, tabv):
    tid = lax.axis_index("c") * NS + lax.axis_index("s")
    pltpu.sync_copy(dst1d.at[pl.ds(tid * EPT, EPT)], dvm)
    pltpu.sync_copy(zeros_hbm, tabv)
    ones = jnp.ones((16,), jnp.float32)

    def body(i, carry):
        for u in range(4):
            iv = dvm[pl.ds((i * 4 + u) * 16, 16)]
            plsc.addupdate_scatter(tabv, [iv], ones)
        return carry

    lax.fori_loop(0, EPT // 64, body, 0)
    iv = dvm[pl.ds(EPT - 16, 16)]
    plsc.addupdate_scatter(tabv, [iv], ones)
    for r in range(N_NODES // 1000):
        pltpu.sync_copy(tabv.at[pl.ds(r * 1000, 1000)],
                        out.at[pl.ds(r * NW * 1000 + tid * 1000, 1000)])


@functools.partial(
    pl.kernel,
    mesh=_mesh,
    out_type=jax.ShapeDtypeStruct((NC, N_NODES, HIDDEN_DIM), jnp.float32),
    scratch_types=[
        pltpu.VMEM((CW,), jnp.int32),
        pltpu.VMEM((CW,), jnp.int32),
        pltpu.VMEM((B, HIDDEN_DIM), jnp.float32),
        pltpu.VMEM((B, HIDDEN_DIM), jnp.float32),
        pltpu.VMEM_SHARED((N_NODES, HIDDEN_DIM), jnp.float32),
        pltpu.SemaphoreType.DMA,
        pltpu.SemaphoreType.DMA,
        pltpu.SemaphoreType.DMA,
        pltpu.SemaphoreType.DMA,
    ],
)
def _sc_spmm(table, src1d, dst1d, zeros_hbm, out,
             svm, dvm, rowsA, rowsB, acc, gA, gB, sA, sB):
    c = lax.axis_index("c")
    s = lax.axis_index("s")
    r0 = s * STRIPE
    @pl.when(c == 0)
    def _():
        pltpu.sync_copy(table.at[pl.ds(r0, STRIPE)], acc.at[pl.ds(r0, STRIPE)])

        @pl.when(s == NS - 1)
        def _():
            pltpu.sync_copy(table.at[pl.ds(NS * STRIPE, STRIPE_REM)],
                            acc.at[pl.ds(NS * STRIPE, STRIPE_REM)])

    @pl.when(c == 1)
    def _():
        pltpu.sync_copy(zeros_hbm.at[pl.ds(r0, STRIPE)],
                        acc.at[pl.ds(r0, STRIPE)])

        @pl.when(s == NS - 1)
        def _():
            pltpu.sync_copy(zeros_hbm.at[pl.ds(NS * STRIPE, STRIPE_REM)],
                            acc.at[pl.ds(NS * STRIPE, STRIPE_REM)])

    w = (c * NS + s) * EPT
    plsc.subcore_barrier()

    def sv(j):
        return svm.at[pl.ds(j * B, B)]

    def dv(j):
        return dvm.at[pl.ds(j * B, B)]

    def chunk(ch, carry):
        wch = w + ch * CW
        pltpu.sync_copy(src1d.at[pl.ds(wch, CW)], svm)
        pltpu.sync_copy(dst1d.at[pl.ds(wch, CW)], dvm)
        pltpu.async_copy(table.at[sv(0)], rowsA, gA)

        def body(i, carry):
            jA = 2 * i
            jB = 2 * i + 1
            pltpu.make_async_copy(table.at[sv(jA)], rowsA, gA).wait()

            @pl.when(i > 0)
            def _():
                pltpu.make_async_copy(rowsB, acc.at[dv(jB)], sB).wait()

            descGB = pltpu.async_copy(table.at[sv(jB)], rowsB, gB)
            descSA = pltpu.async_copy(rowsA, acc.at[dv(jA)], sA, add=True)
            descGB.wait()
            pltpu.async_copy(rowsB, acc.at[dv(jB)], sB, add=True)
            descSA.wait()

            @pl.when(i < CPAIRS - 1)
            def _():
                pltpu.async_copy(table.at[sv(jA + 2)], rowsA, gA)
            return carry

        lax.fori_loop(0, CPAIRS, body, 0)
        pltpu.make_async_copy(rowsB, acc.at[dv(CBT - 1)], sB).wait()
        return carry

    lax.fori_loop(0, CH, chunk, 0)

    pltpu.sync_copy(src1d.at[pl.ds(w + NBT * B, TAIL)], svm.at[pl.ds(0, TAIL)])
    pltpu.sync_copy(dst1d.at[pl.ds(w + NBT * B, TAIL)], dvm.at[pl.ds(0, TAIL)])
    rt = rowsA.at[pl.ds(0, TAIL)]
    pltpu.async_copy(table.at[svm.at[pl.ds(0, TAIL)]], rt, gA).wait()
    pltpu.sync_copy(rt, acc.at[dvm.at[pl.ds(0, TAIL)]], add=True)

    plsc.subcore_barrier()
    pltpu.sync_copy(acc.at[pl.ds(r0, STRIPE)], out.at[c].at[pl.ds(r0, STRIPE)])

    @pl.when(s == NS - 1)
    def _():
        pltpu.sync_copy(acc.at[pl.ds(NS * STRIPE, STRIPE_REM)],
                        out.at[c].at[pl.ds(NS * STRIPE, STRIPE_REM)])


R = 1000
GRID = N_NODES // R
def _dinv_of(degp_ref):
    ones = jnp.ones((NW, HIDDEN_DIM), jnp.float32)
    deg = lax.dot_general(degp_ref[0], ones, (((0,), (0,)), ((), ())),
                          preferred_element_type=jnp.float32)
    return lax.rsqrt(deg + 1.0)


def _tc1_body(x_ref, w_ref, degp_ref, o_ref):
    h0 = jnp.dot(x_ref[...], w_ref[...], preferred_element_type=jnp.float32)
    o_ref[...] = h0 * _dinv_of(degp_ref)


def _tc2_body(s_ref, degp_ref, b_ref, o_ref):
    dinv = _dinv_of(degp_ref)
    h1 = jnp.maximum(dinv * (s_ref[0] + s_ref[1]) + b_ref[...], 0.0)
    o_ref[...] = dinv * h1


def _tc3_body(s_ref, degp_ref, wm_ref, bm_ref, wl_ref, bl_ref,
              mu_ref, lv_ref):
    g = _dinv_of(degp_ref) * (s_ref[0] + s_ref[1])
    mu_ref[...] = jnp.dot(g, wm_ref[...], preferred_element_type=jnp.float32) + bm_ref[...]
    lv_ref[...] = jnp.dot(g, wl_ref[...], preferred_element_type=jnp.float32) + bl_ref[...]


def _row_spec(d):
    return pl.BlockSpec((R, d), lambda i: (i, 0))


def _part_spec(d):
    return pl.BlockSpec((NC, R, d), lambda i: (0, i, 0))


_deg_spec = pl.BlockSpec((1, NW, R), lambda i: (i, 0, 0))


def _full_spec(a, b):
    return pl.BlockSpec((a, b), lambda i: (0, 0))


_tc1 = pl.pallas_call(
    _tc1_body,
    grid=(GRID,),
    in_specs=[_row_spec(IN_DIM), _full_spec(IN_DIM, HIDDEN_DIM), _deg_spec],
    out_specs=_row_spec(HIDDEN_DIM),
    out_shape=jax.ShapeDtypeStruct((N_NODES, HIDDEN_DIM), jnp.float32),
)

_tc2 = pl.pallas_call(
    _tc2_body,
    grid=(GRID,),
    in_specs=[_part_spec(HIDDEN_DIM), _deg_spec,
              _full_spec(1, HIDDEN_DIM)],
    out_specs=_row_spec(HIDDEN_DIM),
    out_shape=jax.ShapeDtypeStruct((N_NODES, HIDDEN_DIM), jnp.float32),
)

_tc3 = pl.pallas_call(
    _tc3_body,
    grid=(GRID,),
    in_specs=[_part_spec(HIDDEN_DIM), _deg_spec,
              _full_spec(HIDDEN_DIM, Z_DIM), _full_spec(1, Z_DIM),
              _full_spec(HIDDEN_DIM, Z_DIM), _full_spec(1, Z_DIM)],
    out_specs=[_row_spec(Z_DIM), _row_spec(Z_DIM)],
    out_shape=[jax.ShapeDtypeStruct((N_NODES, Z_DIM), jnp.float32),
               jax.ShapeDtypeStruct((N_NODES, Z_DIM), jnp.float32)],
)


def kernel(x, edge_index, W1, b1, Wmu, bmu, Wlv, blv):
    src1d = edge_index[0].astype(jnp.int32).reshape(N_EDGES)
    dst1d = edge_index[1].astype(jnp.int32).reshape(N_EDGES)
    z1 = jnp.zeros((N_NODES,), jnp.float32)
    z_h = jnp.zeros((N_NODES, HIDDEN_DIM), jnp.float32)

    degp = _sc_deg(dst1d, z1).reshape(GRID, NW, R)
    h0s = _tc1(x, W1, degp)
    s1 = _sc_spmm(h0s, src1d, dst1d, z_h)
    h1s = _tc2(s1, degp, b1.reshape(1, HIDDEN_DIM))
    s2 = _sc_spmm(h1s, src1d, dst1d, z_h)
    mu, lv = _tc3(s2, degp, Wmu, bmu.reshape(1, Z_DIM),
                  Wlv, blv.reshape(1, Z_DIM))
    return (mu, lv)


# speedup vs baseline: 35.1141x; 1.0501x over previous
import functools

import jax
import jax.numpy as jnp
from jax import lax
from jax.experimental import pallas as pl
from jax.experimental.pallas import tpu as pltpu
from jax.experimental.pallas import tpu_sc as plsc

N_NODES = 10000
N_EDGES = 320000
IN_DIM = 128
HIDDEN_DIM = 128
Z_DIM = 64

B = 64
NBUF = 4
NC, NS = 2, 16
NW = NC * NS
EPT = N_EDGES // NW
NBT = EPT // B
TAIL = EPT - NBT * B
CH = 3
CBT = NBT // CH
CROUNDS = CBT // NBUF
CW = CBT * B
STRIPE = 624
STRIPE_REM = N_NODES - NS * STRIPE

_mesh = plsc.VectorSubcoreMesh(core_axis_name="c", subcore_axis_name="s")


@functools.partial(
    pl.kernel,
    mesh=_mesh,
    compiler_params=pltpu.CompilerParams(needs_layout_passes=False),
    out_type=jax.ShapeDtypeStruct((NW * N_NODES,), jnp.float32),
    scratch_types=[
        pltpu.VMEM((EPT,), jnp.int32),
        pltpu.VMEM((N_NODES,), jnp.float32),
    ],
)
def _sc_deg(dst1d, zeros_hbm, out, dvm, tabv):
    tid = lax.axis_index("c") * NS + lax.axis_index("s")
    pltpu.sync_copy(dst1d.at[pl.ds(tid * EPT, EPT)], dvm)
    pltpu.sync_copy(zeros_hbm, tabv)
    ones = jnp.ones((16,), jnp.float32)

    def body(i, carry):
        for u in range(4):
            iv = dvm[pl.ds((i * 4 + u) * 16, 16)]
            plsc.addupdate_scatter(tabv, [iv], ones)
        return carry

    lax.fori_loop(0, EPT // 64, body, 0)
    iv = dvm[pl.ds(EPT - 16, 16)]
    plsc.addupdate_scatter(tabv, [iv], ones)
    for r in range(N_NODES // 1000):
        pltpu.sync_copy(tabv.at[pl.ds(r * 1000, 1000)],
                        out.at[pl.ds(r * NW * 1000 + tid * 1000, 1000)])


@functools.partial(
    pl.kernel,
    mesh=_mesh,
    out_type=jax.ShapeDtypeStruct((NC, N_NODES, HIDDEN_DIM), jnp.float32),
    scratch_types=[
        pltpu.VMEM((CW,), jnp.int32),
        pltpu.VMEM((CW,), jnp.int32),
        [pltpu.VMEM((B, HIDDEN_DIM), jnp.float32)] * NBUF,
        pltpu.VMEM_SHARED((N_NODES, HIDDEN_DIM), jnp.float32),
        [pltpu.SemaphoreType.DMA] * NBUF,
        [pltpu.SemaphoreType.DMA] * NBUF,
    ],
)
def _sc_spmm(table, src1d, dst1d, zeros_hbm, out,
             svm, dvm, rows, acc, gsem, ssem):
    c = lax.axis_index("c")
    s = lax.axis_index("s")
    r0 = s * STRIPE
    @pl.when(c == 0)
    def _():
        pltpu.sync_copy(table.at[pl.ds(r0, STRIPE)], acc.at[pl.ds(r0, STRIPE)])

        @pl.when(s == NS - 1)
        def _():
            pltpu.sync_copy(table.at[pl.ds(NS * STRIPE, STRIPE_REM)],
                            acc.at[pl.ds(NS * STRIPE, STRIPE_REM)])

    @pl.when(c == 1)
    def _():
        pltpu.sync_copy(zeros_hbm.at[pl.ds(r0, STRIPE)],
                        acc.at[pl.ds(r0, STRIPE)])

        @pl.when(s == NS - 1)
        def _():
            pltpu.sync_copy(zeros_hbm.at[pl.ds(NS * STRIPE, STRIPE_REM)],
                            acc.at[pl.ds(NS * STRIPE, STRIPE_REM)])

    w = (c * NS + s) * EPT
    plsc.subcore_barrier()

    def sv(j):
        return svm.at[pl.ds(j * B, B)]

    def dv(j):
        return dvm.at[pl.ds(j * B, B)]

    def gstart(j, u):
        pltpu.async_copy(table.at[sv(j)], rows[u], gsem[u])

    def gwait(j, u):
        pltpu.make_async_copy(table.at[sv(j)], rows[u], gsem[u]).wait()

    def sstart(j, u):
        pltpu.async_copy(rows[u], acc.at[dv(j)], ssem[u], add=True)

    def swait(j, u):
        pltpu.make_async_copy(rows[u], acc.at[dv(j)], ssem[u]).wait()

    def chunk(ch, carry):
        wch = w + ch * CW
        pltpu.sync_copy(src1d.at[pl.ds(wch, CW)], svm)
        pltpu.sync_copy(dst1d.at[pl.ds(wch, CW)], dvm)
        gstart(0, 0)
        gstart(1, 1)

        def body(r, carry):
            for u in range(NBUF):
                j = NBUF * r + u
                gwait(j, u)
                sstart(j, u)
                wnext = (u + 2) % NBUF
                if u < 2:
                    @pl.when(r > 0)
                    def _():
                        swait(j - 2, wnext)
                    gstart(j + 2, wnext)
                else:
                    swait(j - 2, wnext)

                    @pl.when(r < CROUNDS - 1)
                    def _():
                        gstart(j + 2, wnext)
            return carry

        lax.fori_loop(0, CROUNDS, body, 0)
        swait(CBT - 2, (CBT - 2) % NBUF)
        swait(CBT - 1, (CBT - 1) % NBUF)
        return carry

    lax.fori_loop(0, CH, chunk, 0)

    pltpu.sync_copy(src1d.at[pl.ds(w + NBT * B, TAIL)], svm.at[pl.ds(0, TAIL)])
    pltpu.sync_copy(dst1d.at[pl.ds(w + NBT * B, TAIL)], dvm.at[pl.ds(0, TAIL)])
    rt = rows[0].at[pl.ds(0, TAIL)]
    pltpu.async_copy(table.at[svm.at[pl.ds(0, TAIL)]], rt, gsem[0]).wait()
    pltpu.sync_copy(rt, acc.at[dvm.at[pl.ds(0, TAIL)]], add=True)

    plsc.subcore_barrier()
    pltpu.sync_copy(acc.at[pl.ds(r0, STRIPE)], out.at[c].at[pl.ds(r0, STRIPE)])

    @pl.when(s == NS - 1)
    def _():
        pltpu.sync_copy(acc.at[pl.ds(NS * STRIPE, STRIPE_REM)],
                        out.at[c].at[pl.ds(NS * STRIPE, STRIPE_REM)])


R = 1000
GRID = N_NODES // R
def _dinv_of(degp_ref):
    ones = jnp.ones((NW, HIDDEN_DIM), jnp.float32)
    deg = lax.dot_general(degp_ref[0], ones, (((0,), (0,)), ((), ())),
                          preferred_element_type=jnp.float32)
    return lax.rsqrt(deg + 1.0)


def _tc1_body(x_ref, w_ref, degp_ref, o_ref):
    h0 = jnp.dot(x_ref[...], w_ref[...], preferred_element_type=jnp.float32)
    o_ref[...] = h0 * _dinv_of(degp_ref)


def _tc2_body(s_ref, degp_ref, b_ref, o_ref):
    dinv = _dinv_of(degp_ref)
    h1 = jnp.maximum(dinv * (s_ref[0] + s_ref[1]) + b_ref[...], 0.0)
    o_ref[...] = dinv * h1


def _tc3_body(s_ref, degp_ref, wm_ref, bm_ref, wl_ref, bl_ref,
              mu_ref, lv_ref):
    g = _dinv_of(degp_ref) * (s_ref[0] + s_ref[1])
    mu_ref[...] = jnp.dot(g, wm_ref[...], preferred_element_type=jnp.float32) + bm_ref[...]
    lv_ref[...] = jnp.dot(g, wl_ref[...], preferred_element_type=jnp.float32) + bl_ref[...]


def _row_spec(d):
    return pl.BlockSpec((R, d), lambda i: (i, 0))


def _part_spec(d):
    return pl.BlockSpec((NC, R, d), lambda i: (0, i, 0))


_deg_spec = pl.BlockSpec((1, NW, R), lambda i: (i, 0, 0))


def _full_spec(a, b):
    return pl.BlockSpec((a, b), lambda i: (0, 0))


_tc1 = pl.pallas_call(
    _tc1_body,
    grid=(GRID,),
    in_specs=[_row_spec(IN_DIM), _full_spec(IN_DIM, HIDDEN_DIM), _deg_spec],
    out_specs=_row_spec(HIDDEN_DIM),
    out_shape=jax.ShapeDtypeStruct((N_NODES, HIDDEN_DIM), jnp.float32),
)

_tc2 = pl.pallas_call(
    _tc2_body,
    grid=(GRID,),
    in_specs=[_part_spec(HIDDEN_DIM), _deg_spec,
              _full_spec(1, HIDDEN_DIM)],
    out_specs=_row_spec(HIDDEN_DIM),
    out_shape=jax.ShapeDtypeStruct((N_NODES, HIDDEN_DIM), jnp.float32),
)

_tc3 = pl.pallas_call(
    _tc3_body,
    grid=(GRID,),
    in_specs=[_part_spec(HIDDEN_DIM), _deg_spec,
              _full_spec(HIDDEN_DIM, Z_DIM), _full_spec(1, Z_DIM),
              _full_spec(HIDDEN_DIM, Z_DIM), _full_spec(1, Z_DIM)],
    out_specs=[_row_spec(Z_DIM), _row_spec(Z_DIM)],
    out_shape=[jax.ShapeDtypeStruct((N_NODES, Z_DIM), jnp.float32),
               jax.ShapeDtypeStruct((N_NODES, Z_DIM), jnp.float32)],
)


def kernel(x, edge_index, W1, b1, Wmu, bmu, Wlv, blv):
    src1d = edge_index[0].astype(jnp.int32).reshape(N_EDGES)
    dst1d = edge_index[1].astype(jnp.int32).reshape(N_EDGES)
    z1 = jnp.zeros((N_NODES,), jnp.float32)
    z_h = jnp.zeros((N_NODES, HIDDEN_DIM), jnp.float32)

    degp = _sc_deg(dst1d, z1).reshape(GRID, NW, R)
    h0s = _tc1(x, W1, degp)
    s1 = _sc_spmm(h0s, src1d, dst1d, z_h)
    h1s = _tc2(s1, degp, b1.reshape(1, HIDDEN_DIM))
    s2 = _sc_spmm(h1s, src1d, dst1d, z_h)
    mu, lv = _tc3(s2, degp, Wmu, bmu.reshape(1, Z_DIM),
                  Wlv, blv.reshape(1, Z_DIM))
    return (mu, lv)


# speedup vs baseline: 35.7382x; 1.0178x over previous
import functools

import jax
import jax.numpy as jnp
from jax import lax
from jax.experimental import pallas as pl
from jax.experimental.pallas import tpu as pltpu
from jax.experimental.pallas import tpu_sc as plsc

N_NODES = 10000
N_EDGES = 320000
IN_DIM = 128
HIDDEN_DIM = 128
Z_DIM = 64

B = 64
NBUF = 4
NC, NS = 2, 16
NW = NC * NS
EPT = N_EDGES // NW
NBT = EPT // B
TAIL = EPT - NBT * B
CH = 3
CBT = NBT // CH
CROUNDS = CBT // NBUF
CW = CBT * B
STRIPE = 624
STRIPE_REM = N_NODES - NS * STRIPE

_mesh = plsc.VectorSubcoreMesh(core_axis_name="c", subcore_axis_name="s")


@functools.partial(
    pl.kernel,
    mesh=_mesh,
    compiler_params=pltpu.CompilerParams(needs_layout_passes=False),
    out_type=jax.ShapeDtypeStruct((NW * N_NODES,), jnp.float32),
    scratch_types=[
        pltpu.VMEM((EPT,), jnp.int32),
        pltpu.VMEM((N_NODES,), jnp.float32),
    ],
)
def _sc_deg(dst1d, zeros_hbm, out, dvm, tabv):
    tid = lax.axis_index("c") * NS + lax.axis_index("s")
    pltpu.sync_copy(dst1d.at[pl.ds(tid * EPT, EPT)], dvm)
    pltpu.sync_copy(zeros_hbm, tabv)
    ones = jnp.ones((16,), jnp.float32)

    def body(i, carry):
        for u in range(4):
            iv = dvm[pl.ds((i * 4 + u) * 16, 16)]
            plsc.addupdate_scatter(tabv, [iv], ones)
        return carry

    lax.fori_loop(0, EPT // 64, body, 0)
    iv = dvm[pl.ds(EPT - 16, 16)]
    plsc.addupdate_scatter(tabv, [iv], ones)
    for r in range(N_NODES // 1000):
        pltpu.sync_copy(tabv.at[pl.ds(r * 1000, 1000)],
                        out.at[pl.ds(r * NW * 1000 + tid * 1000, 1000)])


@functools.partial(
    pl.kernel,
    mesh=_mesh,
    out_type=jax.ShapeDtypeStruct((NC, N_NODES, HIDDEN_DIM), jnp.float32),
    scratch_types=[
        [pltpu.VMEM((CW,), jnp.int32)] * 2,
        [pltpu.VMEM((CW,), jnp.int32)] * 2,
        [pltpu.VMEM((B, HIDDEN_DIM), jnp.float32)] * NBUF,
        pltpu.VMEM_SHARED((N_NODES, HIDDEN_DIM), jnp.float32),
        [pltpu.SemaphoreType.DMA] * NBUF,
        [pltpu.SemaphoreType.DMA] * NBUF,
        pltpu.SemaphoreType.DMA,
    ],
)
def _sc_spmm(table, src1d, dst1d, zeros_hbm, out,
             svm, dvm, rows, acc, gsem, ssem, psem):
    c = lax.axis_index("c")
    s = lax.axis_index("s")
    r0 = s * STRIPE
    @pl.when(c == 0)
    def _():
        pltpu.sync_copy(table.at[pl.ds(r0, STRIPE)], acc.at[pl.ds(r0, STRIPE)])

        @pl.when(s == NS - 1)
        def _():
            pltpu.sync_copy(table.at[pl.ds(NS * STRIPE, STRIPE_REM)],
                            acc.at[pl.ds(NS * STRIPE, STRIPE_REM)])

    @pl.when(c == 1)
    def _():
        pltpu.sync_copy(zeros_hbm.at[pl.ds(r0, STRIPE)],
                        acc.at[pl.ds(r0, STRIPE)])

        @pl.when(s == NS - 1)
        def _():
            pltpu.sync_copy(zeros_hbm.at[pl.ds(NS * STRIPE, STRIPE_REM)],
                            acc.at[pl.ds(NS * STRIPE, STRIPE_REM)])

    w = (c * NS + s) * EPT
    plsc.subcore_barrier()

    def sv(j):
        return svm.at[pl.ds(j * B, B)]

    def dv(j):
        return dvm.at[pl.ds(j * B, B)]

    def gstart(p, j, u):
        pltpu.async_copy(table.at[svm[p].at[pl.ds(j * B, B)]], rows[u], gsem[u])

    def gwait(p, j, u):
        pltpu.make_async_copy(table.at[svm[p].at[pl.ds(j * B, B)]],
                              rows[u], gsem[u]).wait()

    def sstart(p, j, u):
        pltpu.async_copy(rows[u], acc.at[dvm[p].at[pl.ds(j * B, B)]],
                         ssem[u], add=True)

    def swait(p, j, u):
        pltpu.make_async_copy(rows[u], acc.at[dvm[p].at[pl.ds(j * B, B)]],
                              ssem[u]).wait()

    pltpu.sync_copy(src1d.at[pl.ds(w, CW)], svm[0])
    pltpu.sync_copy(dst1d.at[pl.ds(w, CW)], dvm[0])
    gstart(0, 0, 0)
    gstart(0, 1, 1)

    for ck in range(CH):
        p = ck % 2
        q = 1 - p
        for u in range(NBUF):
            gwait(p, u, u)
            sstart(p, u, u)
            wn = (u + 2) % NBUF
            if u < 2:
                if ck > 0:
                    swait(1 - p, CBT - 2 + u, wn)
            else:
                swait(p, u - 2, wn)
            gstart(p, u + 2, wn)
        if ck < CH - 1:
            wn_ = w + (ck + 1) * CW
            pltpu.async_copy(src1d.at[pl.ds(wn_, CW)], svm[q], psem)
            pltpu.async_copy(dst1d.at[pl.ds(wn_, CW)], dvm[q], psem)

        def body(r, carry):
            for u in range(NBUF):
                j = NBUF * r + u
                gwait(p, j, u)
                sstart(p, j, u)
                wn = (u + 2) % NBUF
                swait(p, j - 2, wn)
                gstart(p, j + 2, wn)
            return carry

        lax.fori_loop(1, CROUNDS - 1, body, 0)

        for u in range(NBUF):
            j = CBT - NBUF + u
            gwait(p, j, u)
            sstart(p, j, u)
            wn = (u + 2) % NBUF
            swait(p, j - 2, wn)
            if u < 2:
                gstart(p, j + 2, wn)
            elif ck < CH - 1:
                if u == 2:
                    pltpu.make_async_copy(src1d.at[pl.ds(0, CW)], svm[q],
                                          psem).wait()
                    pltpu.make_async_copy(dst1d.at[pl.ds(0, CW)], dvm[q],
                                          psem).wait()
                gstart(q, u - 2, wn)

    pf = (CH - 1) % 2
    swait(pf, CBT - 2, (CBT - 2) % NBUF)
    swait(pf, CBT - 1, (CBT - 1) % NBUF)

    pltpu.sync_copy(src1d.at[pl.ds(w + NBT * B, TAIL)], svm[0].at[pl.ds(0, TAIL)])
    pltpu.sync_copy(dst1d.at[pl.ds(w + NBT * B, TAIL)], dvm[0].at[pl.ds(0, TAIL)])
    rt = rows[0].at[pl.ds(0, TAIL)]
    pltpu.async_copy(table.at[svm[0].at[pl.ds(0, TAIL)]], rt, gsem[0]).wait()
    pltpu.sync_copy(rt, acc.at[dvm[0].at[pl.ds(0, TAIL)]], add=True)

    plsc.subcore_barrier()
    pltpu.sync_copy(acc.at[pl.ds(r0, STRIPE)], out.at[c].at[pl.ds(r0, STRIPE)])

    @pl.when(s == NS - 1)
    def _():
        pltpu.sync_copy(acc.at[pl.ds(NS * STRIPE, STRIPE_REM)],
                        out.at[c].at[pl.ds(NS * STRIPE, STRIPE_REM)])


R = 1000
GRID = N_NODES // R
def _dinv_of(degp_ref):
    ones = jnp.ones((NW, HIDDEN_DIM), jnp.float32)
    deg = lax.dot_general(degp_ref[0], ones, (((0,), (0,)), ((), ())),
                          preferred_element_type=jnp.float32)
    return lax.rsqrt(deg + 1.0)


def _tc1_body(x_ref, w_ref, degp_ref, o_ref):
    h0 = jnp.dot(x_ref[...], w_ref[...], preferred_element_type=jnp.float32)
    o_ref[...] = h0 * _dinv_of(degp_ref)


def _tc2_body(s_ref, degp_ref, b_ref, o_ref):
    dinv = _dinv_of(degp_ref)
    h1 = jnp.maximum(dinv * (s_ref[0] + s_ref[1]) + b_ref[...], 0.0)
    o_ref[...] = dinv * h1


def _tc3_body(s_ref, degp_ref, wm_ref, bm_ref, wl_ref, bl_ref,
              mu_ref, lv_ref):
    g = _dinv_of(degp_ref) * (s_ref[0] + s_ref[1])
    mu_ref[...] = jnp.dot(g, wm_ref[...], preferred_element_type=jnp.float32) + bm_ref[...]
    lv_ref[...] = jnp.dot(g, wl_ref[...], preferred_element_type=jnp.float32) + bl_ref[...]


def _row_spec(d):
    return pl.BlockSpec((R, d), lambda i: (i, 0))


def _part_spec(d):
    return pl.BlockSpec((NC, R, d), lambda i: (0, i, 0))


_deg_spec = pl.BlockSpec((1, NW, R), lambda i: (i, 0, 0))


def _full_spec(a, b):
    return pl.BlockSpec((a, b), lambda i: (0, 0))


_tc1 = pl.pallas_call(
    _tc1_body,
    grid=(GRID,),
    in_specs=[_row_spec(IN_DIM), _full_spec(IN_DIM, HIDDEN_DIM), _deg_spec],
    out_specs=_row_spec(HIDDEN_DIM),
    out_shape=jax.ShapeDtypeStruct((N_NODES, HIDDEN_DIM), jnp.float32),
)

_tc2 = pl.pallas_call(
    _tc2_body,
    grid=(GRID,),
    in_specs=[_part_spec(HIDDEN_DIM), _deg_spec,
              _full_spec(1, HIDDEN_DIM)],
    out_specs=_row_spec(HIDDEN_DIM),
    out_shape=jax.ShapeDtypeStruct((N_NODES, HIDDEN_DIM), jnp.float32),
)

_tc3 = pl.pallas_call(
    _tc3_body,
    grid=(GRID,),
    in_specs=[_part_spec(HIDDEN_DIM), _deg_spec,
              _full_spec(HIDDEN_DIM, Z_DIM), _full_spec(1, Z_DIM),
              _full_spec(HIDDEN_DIM, Z_DIM), _full_spec(1, Z_DIM)],
    out_specs=[_row_spec(Z_DIM), _row_spec(Z_DIM)],
    out_shape=[jax.ShapeDtypeStruct((N_NODES, Z_DIM), jnp.float32),
               jax.ShapeDtypeStruct((N_NODES, Z_DIM), jnp.float32)],
)


def kernel(x, edge_index, W1, b1, Wmu, bmu, Wlv, blv):
    src1d = edge_index[0].astype(jnp.int32).reshape(N_EDGES)
    dst1d = edge_index[1].astype(jnp.int32).reshape(N_EDGES)
    z1 = jnp.zeros((N_NODES,), jnp.float32)
    z_h = jnp.zeros((N_NODES, HIDDEN_DIM), jnp.float32)

    degp = _sc_deg(dst1d, z1).reshape(GRID, NW, R)
    h0s = _tc1(x, W1, degp)
    s1 = _sc_spmm(h0s, src1d, dst1d, z_h)
    h1s = _tc2(s1, degp, b1.reshape(1, HIDDEN_DIM))
    s2 = _sc_spmm(h1s, src1d, dst1d, z_h)
    mu, lv = _tc3(s2, degp, Wmu, bmu.reshape(1, Z_DIM),
                  Wlv, blv.reshape(1, Z_DIM))
    return (mu, lv)
